# baseline (speedup 1.0000x reference)
import functools

import jax
import jax.numpy as jnp
from jax import lax
from jax.experimental import pallas as pl
from jax.experimental.pallas import tpu as pltpu
from jax.experimental.pallas import tpu_sc as plsc

B = 16384
D = 64
L = 16
NC = 2
NS = 16
NW = NC * NS
NWINDOWS = 7813
LASTWIN = NWINDOWS - 1
MCAP = 2048
HCAP = 64
DUMP = B
ROWS = B + 128

_mesh = plsc.VectorSubcoreMesh(core_axis_name="c", subcore_axis_name="s")


@functools.partial(
    pl.kernel,
    mesh=_mesh,
    compiler_params=pltpu.CompilerParams(
        needs_layout_passes=False, use_tc_tiling_on_sc=True),
    out_type=(jax.ShapeDtypeStruct((ROWS, 128), jnp.float32),
              jax.ShapeDtypeStruct((ROWS, 128), jnp.float32)),
    scratch_types=[
        pltpu.VMEM((B,), jnp.int32),
        pltpu.VMEM((MCAP,), jnp.int32),
        pltpu.VMEM((MCAP,), jnp.int32),
        pltpu.VMEM((HCAP,), jnp.int32),
        pltpu.VMEM((HCAP,), jnp.int32),
        pltpu.VMEM((256,), jnp.int32),
        pltpu.VMEM((256,), jnp.int32),
        pltpu.VMEM((256,), jnp.int32),
        pltpu.VMEM((272,), jnp.int32),
        pltpu.VMEM((D, 128), jnp.float32),
        pltpu.VMEM((D, 128), jnp.float32),
        pltpu.VMEM((D, 128), jnp.float32),
        pltpu.VMEM((D, 128), jnp.float32),
        pltpu.VMEM((D, 128), jnp.float32),
        pltpu.VMEM((D, 128), jnp.float32),
        pltpu.VMEM((16, 128), jnp.float32),
        pltpu.VMEM((16,), jnp.int32),
        pltpu.SemaphoreType.DMA,
        pltpu.SemaphoreType.DMA,
        pltpu.SemaphoreType.DMA,
        pltpu.SemaphoreType.DMA,
        pltpu.SemaphoreType.DMA,
        pltpu.SemaphoreType.DMA,
        pltpu.SemaphoreType.DMA,
    ],
)
def _sweep_sc(uids_hbm, iids_hbm, pu_hbm, pv_hbm, ug_hbm, vg_hbm,
              idb, moff, mpos, hid, hpo, soff, spos, wcnts, wlist, w0b, w1b, w2b, w3b, w4b, w5b,
              stage, posb, sem0, sem1, sem2, sem3, sem4, sem5, ssem):
    wid = lax.axis_index("s") * NC + lax.axis_index("c")
    W0 = jnp.where(wid < 5, 245 * wid, 244 * wid + 5)
    NWIN = jnp.where(wid < 5, 245, 244)
    lanes = lax.iota(jnp.int32, L)
    lane0 = lanes == 0
    wbufs = (w0b, w1b, w2b, w3b, w4b, w5b)
    wsems = (sem0, sem1, sem2, sem3, sem4, sem5)

    zero16 = jnp.zeros((L,), jnp.float32)
    for r in range(16):
        for c in range(4, 8):
            plsc.store_scatter(
                stage, [jnp.full((L,), r, jnp.int32),
                        c * L + lanes], zero16)
    posb[...] = jnp.full((L,), DUMP, jnp.int32)

    def do_table(tab_hbm, ids_hbm, out_hbm):
        pltpu.sync_copy(ids_hbm, idb)
        lo128 = W0 * 128
        hi128 = (W0 + NWIN) * 128

        def a_body(g, cnt):
            ids16 = idb[pl.ds(g * L, L)]
            m = (ids16 >= lo128) & (ids16 < hi128)
            plsc.store_compressed(moff.at[pl.ds(cnt, L)], ids16, mask=m)
            plsc.store_compressed(
                mpos.at[pl.ds(cnt, L)], g * L + lanes, mask=m)
            c = plsc.all_reduce_population_count(m)[0]
            return jnp.minimum(cnt + c, MCAP - L)

        cnt = lax.fori_loop(0, B // L, a_body, jnp.int32(0))
        ngroups = (cnt + L - 1) // L

        zeros16i = jnp.zeros((L,), jnp.int32)
        for i in range(16):
            wcnts[pl.ds(i * L, L)] = zeros16i
        ones16 = jnp.full((L,), 1, jnp.int32)

        def h_body(gi, _):
            i16 = moff[pl.ds(gi * L, L)]
            valid = (gi * L + lanes) < cnt
            plsc.addupdate_scatter(
                wcnts, [(i16 >> 7) - W0], ones16, mask=valid)
            return 0

        lax.fori_loop(0, ngroups, h_body, 0)

        def l_body(gi, wc):
            widx16 = gi * L + lanes
            c16 = wcnts[pl.ds(gi * L, L)]
            m = (c16 > 0) & (widx16 < NWIN)
            plsc.store_compressed(wlist.at[pl.ds(wc, L)], widx16, mask=m)
            return wc + plsc.all_reduce_population_count(m)[0]

        wcount = lax.fori_loop(0, 16, l_body, jnp.int32(0))

        def entry(j):
            jj = jnp.maximum(jnp.minimum(j, wcount - 1), 0)
            return wlist[pl.ds(jj, L)][0]

        def fire_entry(j, slot):
            woff = jnp.minimum(W0 + entry(j), LASTWIN) * 128
            woff = pl.multiple_of(woff, 128)
            return pltpu.async_copy(
                tab_hbm.at[:, pl.ds(woff, 128)], wbufs[slot], wsems[slot])

        for s in range(6):
            fire_entry(jnp.int32(s), s)

        def flush_stage(ss):
            pv = posb[...]
            pltpu.async_copy(stage, out_hbm.at[pv], ssem).wait()
            posb[...] = jnp.full((L,), DUMP, jnp.int32)
            return ss

        def super_filter(w4):
            e0 = w4 * 6
            sw_lo = W0 + entry(e0)
            sw_hi = W0 + entry(e0 + 17)

            def sscan(gi, sc):
                i16 = moff[pl.ds(gi * L, L)]
                p16 = mpos[pl.ds(gi * L, L)]
                valid = (gi * L + lanes) < cnt
                wg = i16 >> 7
                m = (wg >= sw_lo) & (wg <= sw_hi) & valid
                plsc.store_compressed(soff.at[pl.ds(sc, L)], i16, mask=m)
                plsc.store_compressed(spos.at[pl.ds(sc, L)], p16, mask=m)
                c = plsc.all_reduce_population_count(m)[0]
                return jnp.minimum(sc + c, 256 - L)

            return lax.fori_loop(0, ngroups, sscan, jnp.int32(0))

        def process_window(j, k, ss, scnt):
            g = W0 + entry(j)

            def scan(gi, hcnt):
                i16 = soff[pl.ds(gi * L, L)]
                p16 = spos[pl.ds(gi * L, L)]
                valid = (gi * L + lanes) < scnt
                m = ((i16 >> 7) == g) & valid
                plsc.store_compressed(hid.at[pl.ds(hcnt, L)], i16, mask=m)
                plsc.store_compressed(hpo.at[pl.ds(hcnt, L)], p16, mask=m)
                c = plsc.all_reduce_population_count(m)[0]
                return jnp.minimum(hcnt + c, HCAP - L)

            sgroups = (scnt + L - 1) // L
            hcnt = lax.fori_loop(0, sgroups, scan, jnp.int32(0))

            def per_id(i, ss):
                idv = hid[pl.ds(i, L)][0]
                pos = hpo[pl.ds(i, L)][0]
                xs = jnp.full((L,), idv & 127, jnp.int32)
                slot = ss % 16
                rows16 = jnp.full((L,), slot, jnp.int32)
                for c in range(4):
                    vals = plsc.load_gather(wbufs[k], [c * L + lanes, xs])
                    plsc.store_scatter(stage, [rows16, c * L + lanes], vals)
                plsc.store_scatter(
                    posb, [rows16], jnp.full((L,), pos, jnp.int32),
                    mask=lane0)
                return lax.cond(slot == 15, flush_stage,
                                lambda s: s, ss) + 1

            return lax.fori_loop(0, hcnt, per_id, ss)

        def w_body(w4, carry):
            ss, scnt = carry
            scnt = lax.cond(w4 % 3 == 0,
                            lambda: super_filter(w4),
                            lambda: scnt)
            for k in range(6):
                j = w4 * 6 + k
                pltpu.make_async_copy(
                    tab_hbm.at[:, pl.ds(0, 128)], wbufs[k],
                    wsems[k]).wait()
                ss = lax.cond(
                    j < wcount,
                    lambda s, j=j, k=k: process_window(j, k, s, scnt),
                    lambda s: s, ss)
                fire_entry(j + 6, k)
            return ss, scnt

        nouter = (wcount + 5) // 6
        ss, _ = lax.fori_loop(
            0, nouter, w_body, (jnp.int32(0), jnp.int32(0)))
        for k in range(6):
            pltpu.make_async_copy(
                tab_hbm.at[:, pl.ds(0, 128)], wbufs[k], wsems[k]).wait()
        flush_stage(ss)

    do_table(pu_hbm, uids_hbm, ug_hbm)
    do_table(pv_hbm, iids_hbm, vg_hbm)


BPW = B // NW


@functools.partial(
    pl.kernel,
    mesh=_mesh,
    compiler_params=pltpu.CompilerParams(
        needs_layout_passes=False, use_tc_tiling_on_sc=True),
    out_type=jax.ShapeDtypeStruct((B,), jnp.float32),
    scratch_types=[
        pltpu.VMEM((80,), jnp.float32),
        pltpu.VMEM((128, 128), jnp.float32),
        pltpu.VMEM((128, 128), jnp.float32),
        pltpu.VMEM((128, 128), jnp.float32),
        pltpu.VMEM((128, 128), jnp.float32),
        pltpu.VMEM((BPW,), jnp.float32),
        pltpu.SemaphoreType.DMA,
        pltpu.SemaphoreType.DMA,
    ],
)
def _dot_sc(ug_hbm, vg_hbm, wb_hbm, out_hbm,
            wb_s, u0, u1, v0, v1, outb, semu, semv):
    wid = lax.axis_index("s") * NC + lax.axis_index("c")
    base = wid * BPW
    pltpu.sync_copy(wb_hbm, wb_s)
    ubufs = (u0, u1)
    vbufs = (v0, v1)

    def fire(c):
        r0 = pl.multiple_of(base + c * 128, 128)
        cu = pltpu.async_copy(
            ug_hbm.at[pl.ds(r0, 128), :], ubufs[c % 2], semu)
        cv = pltpu.async_copy(
            vg_hbm.at[pl.ds(r0, 128), :], vbufs[c % 2], semv)
        return cu, cv

    cps = [fire(0), fire(1)]
    wc = [wb_s[pl.ds(16 * c, L)] for c in range(4)]
    bias = wb_s[pl.ds(64, L)][0]
    lanes = lax.iota(jnp.int32, L)

    for c in range(4):
        for cp in cps[c % 2]:
            cp.wait()
        if c + 2 < 4:
            cps[c % 2] = fire(c + 2)

        def group_body(g, _, c=c):
            rows = g * L + lanes
            acc = jnp.zeros((L,), jnp.float32)
            for d in range(D):
                dims = jnp.full((L,), d, jnp.int32)
                uv = plsc.load_gather(ubufs[c % 2], [rows, dims])
                vv = plsc.load_gather(vbufs[c % 2], [rows, dims])
                acc = acc + uv * vv * wc[d // L][d % L]
            outb[pl.ds(c * 128 + g * L, L)] = acc + bias
            return 0

        lax.fori_loop(0, 8, group_body, 0)

    pltpu.sync_copy(outb, out_hbm.at[pl.ds(base, BPW)])


def kernel(user_ids, item_ids, user_table, item_table, W, b):
    uids = user_ids.astype(jnp.int32)
    iids = item_ids.astype(jnp.int32)
    ug, vg = _sweep_sc(uids, iids, user_table.T, item_table.T)
    wb = jnp.zeros((80,), jnp.float32).at[:D].set(W[:, 0]).at[D].set(b[0])
    return _dot_sc(ug, vg, wb)


# speedup vs baseline: 1.0034x; 1.0034x over previous
import functools

import jax
import jax.numpy as jnp
from jax import lax
from jax.experimental import pallas as pl
from jax.experimental.pallas import tpu as pltpu
from jax.experimental.pallas import tpu_sc as plsc

B = 16384
D = 64
L = 16
NC = 2
NS = 16
NW = NC * NS
NWINDOWS = 7813
LASTWIN = NWINDOWS - 1
MCAP = 2048
HCAP = 64
DUMP = B
ROWS = B + 128

_mesh = plsc.VectorSubcoreMesh(core_axis_name="c", subcore_axis_name="s")


@functools.partial(
    pl.kernel,
    mesh=_mesh,
    compiler_params=pltpu.CompilerParams(
        needs_layout_passes=False, use_tc_tiling_on_sc=True),
    out_type=(jax.ShapeDtypeStruct((ROWS, 128), jnp.float32),
              jax.ShapeDtypeStruct((ROWS, 128), jnp.float32)),
    scratch_types=[
        pltpu.VMEM((B,), jnp.int32),
        pltpu.VMEM((B,), jnp.int32),
        pltpu.VMEM((MCAP,), jnp.int32),
        pltpu.VMEM((MCAP,), jnp.int32),
        pltpu.VMEM((MCAP,), jnp.int32),
        pltpu.VMEM((MCAP,), jnp.int32),
        pltpu.VMEM((HCAP,), jnp.int32),
        pltpu.VMEM((HCAP,), jnp.int32),
        pltpu.VMEM((256,), jnp.int32),
        pltpu.VMEM((256,), jnp.int32),
        pltpu.VMEM((256,), jnp.int32),
        pltpu.VMEM((272,), jnp.int32),
        pltpu.VMEM((D, 128), jnp.float32),
        pltpu.VMEM((D, 128), jnp.float32),
        pltpu.VMEM((D, 128), jnp.float32),
        pltpu.VMEM((D, 128), jnp.float32),
        pltpu.VMEM((D, 128), jnp.float32),
        pltpu.VMEM((D, 128), jnp.float32),
        pltpu.VMEM((16, 128), jnp.float32),
        pltpu.VMEM((16,), jnp.int32),
        pltpu.SemaphoreType.DMA,
        pltpu.SemaphoreType.DMA,
        pltpu.SemaphoreType.DMA,
        pltpu.SemaphoreType.DMA,
        pltpu.SemaphoreType.DMA,
        pltpu.SemaphoreType.DMA,
        pltpu.SemaphoreType.DMA,
    ],
)
def _sweep_sc(uids_hbm, iids_hbm, pu_hbm, pv_hbm, ug_hbm, vg_hbm,
              idb, idb2, moff, mpos, moff2, mpos2, hid, hpo, soff, spos,
              wcnts, wlist, w0b, w1b, w2b, w3b, w4b, w5b,
              stage, posb, sem0, sem1, sem2, sem3, sem4, sem5, ssem):
    wid = lax.axis_index("s") * NC + lax.axis_index("c")
    W0 = jnp.where(wid < 5, 245 * wid, 244 * wid + 5)
    NWIN = jnp.where(wid < 5, 245, 244)
    lanes = lax.iota(jnp.int32, L)
    lane0 = lanes == 0
    wbufs = (w0b, w1b, w2b, w3b, w4b, w5b)
    wsems = (sem0, sem1, sem2, sem3, sem4, sem5)

    zero16 = jnp.zeros((L,), jnp.float32)
    for r in range(16):
        for c in range(4, 8):
            plsc.store_scatter(
                stage, [jnp.full((L,), r, jnp.int32),
                        c * L + lanes], zero16)
    posb[...] = jnp.full((L,), DUMP, jnp.int32)

    pltpu.sync_copy(uids_hbm, idb)
    pltpu.sync_copy(iids_hbm, idb2)
    lo128 = W0 * 128
    hi128 = (W0 + NWIN) * 128

    def make_a_body(idb_r, moff_r, mpos_r):
        def a_body(g, cnt):
            ids16 = idb_r[pl.ds(g * L, L)]
            m = (ids16 >= lo128) & (ids16 < hi128)
            plsc.store_compressed(moff_r.at[pl.ds(cnt, L)], ids16, mask=m)
            plsc.store_compressed(
                mpos_r.at[pl.ds(cnt, L)], g * L + lanes, mask=m)
            c = plsc.all_reduce_population_count(m)[0]
            return jnp.minimum(cnt + c, MCAP - L)
        return a_body

    a2_body = make_a_body(idb2, moff2, mpos2)

    def do_table(tab_hbm, out_hbm, moff, mpos, cnt, inter):
        ngroups = (cnt + L - 1) // L

        zeros16i = jnp.zeros((L,), jnp.int32)
        for i in range(16):
            wcnts[pl.ds(i * L, L)] = zeros16i
        ones16 = jnp.full((L,), 1, jnp.int32)

        def h_body(gi, _):
            i16 = moff[pl.ds(gi * L, L)]
            valid = (gi * L + lanes) < cnt
            plsc.addupdate_scatter(
                wcnts, [(i16 >> 7) - W0], ones16, mask=valid)
            return 0

        lax.fori_loop(0, ngroups, h_body, 0)

        def l_body(gi, wc):
            widx16 = gi * L + lanes
            c16 = wcnts[pl.ds(gi * L, L)]
            m = (c16 > 0) & (widx16 < NWIN)
            plsc.store_compressed(wlist.at[pl.ds(wc, L)], widx16, mask=m)
            return wc + plsc.all_reduce_population_count(m)[0]

        wcount = lax.fori_loop(0, 16, l_body, jnp.int32(0))

        def entry(j):
            jj = jnp.maximum(jnp.minimum(j, wcount - 1), 0)
            return wlist[pl.ds(jj, L)][0]

        def fire_entry(j, slot):
            woff = jnp.minimum(W0 + entry(j), LASTWIN) * 128
            woff = pl.multiple_of(woff, 128)
            return pltpu.async_copy(
                tab_hbm.at[:, pl.ds(woff, 128)], wbufs[slot], wsems[slot])

        for s in range(6):
            fire_entry(jnp.int32(s), s)

        def flush_stage(ss):
            pv = posb[...]
            pltpu.async_copy(stage, out_hbm.at[pv], ssem).wait()
            posb[...] = jnp.full((L,), DUMP, jnp.int32)
            return ss

        def super_filter(w4):
            e0 = w4 * 6
            sw_lo = W0 + entry(e0)
            sw_hi = W0 + entry(e0 + 17)

            def sscan(gi, sc):
                i16 = moff[pl.ds(gi * L, L)]
                p16 = mpos[pl.ds(gi * L, L)]
                valid = (gi * L + lanes) < cnt
                wg = i16 >> 7
                m = (wg >= sw_lo) & (wg <= sw_hi) & valid
                plsc.store_compressed(soff.at[pl.ds(sc, L)], i16, mask=m)
                plsc.store_compressed(spos.at[pl.ds(sc, L)], p16, mask=m)
                c = plsc.all_reduce_population_count(m)[0]
                return jnp.minimum(sc + c, 256 - L)

            return lax.fori_loop(0, ngroups, sscan, jnp.int32(0))

        def process_window(j, k, ss, scnt):
            g = W0 + entry(j)

            def scan(gi, hcnt):
                i16 = soff[pl.ds(gi * L, L)]
                p16 = spos[pl.ds(gi * L, L)]
                valid = (gi * L + lanes) < scnt
                m = ((i16 >> 7) == g) & valid
                plsc.store_compressed(hid.at[pl.ds(hcnt, L)], i16, mask=m)
                plsc.store_compressed(hpo.at[pl.ds(hcnt, L)], p16, mask=m)
                c = plsc.all_reduce_population_count(m)[0]
                return jnp.minimum(hcnt + c, HCAP - L)

            sgroups = (scnt + L - 1) // L
            hcnt = lax.fori_loop(0, sgroups, scan, jnp.int32(0))

            def per_id(i, ss):
                idv = hid[pl.ds(i, L)][0]
                pos = hpo[pl.ds(i, L)][0]
                xs = jnp.full((L,), idv & 127, jnp.int32)
                slot = ss % 16
                rows16 = jnp.full((L,), slot, jnp.int32)
                for c in range(4):
                    vals = plsc.load_gather(wbufs[k], [c * L + lanes, xs])
                    plsc.store_scatter(stage, [rows16, c * L + lanes], vals)
                plsc.store_scatter(
                    posb, [rows16], jnp.full((L,), pos, jnp.int32),
                    mask=lane0)
                return lax.cond(slot == 15, flush_stage,
                                lambda s: s, ss) + 1

            return lax.fori_loop(0, hcnt, per_id, ss)

        def w_body(w4, carry):
            ss, scnt, c2 = carry
            scnt = lax.cond(w4 % 3 == 0,
                            lambda: super_filter(w4),
                            lambda: scnt)
            c2 = inter(w4, c2)
            for k in range(6):
                j = w4 * 6 + k
                pltpu.make_async_copy(
                    tab_hbm.at[:, pl.ds(0, 128)], wbufs[k],
                    wsems[k]).wait()
                ss = lax.cond(
                    j < wcount,
                    lambda s, j=j, k=k: process_window(j, k, s, scnt),
                    lambda s: s, ss)
                fire_entry(j + 6, k)
            return ss, scnt, c2

        nouter = (wcount + 5) // 6
        ss, _, c2 = lax.fori_loop(
            0, nouter, w_body,
            (jnp.int32(0), jnp.int32(0), jnp.int32(0)))
        for k in range(6):
            pltpu.make_async_copy(
                tab_hbm.at[:, pl.ds(0, 128)], wbufs[k], wsems[k]).wait()
        flush_stage(ss)
        return nouter, c2

    cnt_u = lax.fori_loop(
        0, B // L, make_a_body(idb, moff, mpos), jnp.int32(0))

    AGPI = 24

    def inter_u(w4, c2):
        base = w4 * AGPI
        nb = jnp.clip(B // L - base, 0, AGPI)
        return lax.fori_loop(
            0, nb, lambda g2, cc: a2_body(base + g2, cc), c2)

    nouter_u, cnt2 = do_table(pu_hbm, ug_hbm, moff, mpos, cnt_u, inter_u)
    cnt2 = lax.fori_loop(
        jnp.minimum(nouter_u * AGPI, B // L), B // L, a2_body, cnt2)
    do_table(pv_hbm, vg_hbm, moff2, mpos2, cnt2,
             lambda w4, c2: c2)


def _dot_body(u_ref, v_ref, w_ref, b_ref, o_ref):
    h = u_ref[...] * v_ref[...]
    o_ref[...] = lax.dot_general(
        h, w_ref[...], (((1,), (0,)), ((), ())),
        preferred_element_type=jnp.float32) + b_ref[...]


_dot_tc = pl.pallas_call(
    _dot_body,
    grid=(4,),
    in_specs=[
        pl.BlockSpec((4096, 128), lambda i: (i, 0)),
        pl.BlockSpec((4096, 128), lambda i: (i, 0)),
        pl.BlockSpec((128, 1), lambda i: (0, 0)),
        pl.BlockSpec((1, 1), lambda i: (0, 0)),
    ],
    out_specs=pl.BlockSpec((4096, 1), lambda i: (i, 0)),
    out_shape=jax.ShapeDtypeStruct((B, 1), jnp.float32),
)


def kernel(user_ids, item_ids, user_table, item_table, W, b):
    uids = user_ids.astype(jnp.int32)
    iids = item_ids.astype(jnp.int32)
    ug, vg = _sweep_sc(uids, iids, user_table.T, item_table.T)
    wpad = jnp.zeros((128, 1), jnp.float32).at[:D, 0].set(W[:, 0])
    out2 = _dot_tc(ug, vg, wpad, b.reshape(1, 1))
    return out2[:, 0]


# speedup vs baseline: 1.1646x; 1.1607x over previous
import functools

import jax
import jax.numpy as jnp
from jax import lax
from jax.experimental import pallas as pl
from jax.experimental.pallas import tpu as pltpu
from jax.experimental.pallas import tpu_sc as plsc

B = 16384
D = 64
L = 16
NC = 2
NS = 16
NW = NC * NS
NWINDOWS = 7813
LASTWIN = NWINDOWS - 1
MCAP = 2048
HCAP = 64
DUMP = B
ROWS = B + 128

_mesh = plsc.VectorSubcoreMesh(core_axis_name="c", subcore_axis_name="s")


@functools.partial(
    pl.kernel,
    mesh=_mesh,
    compiler_params=pltpu.CompilerParams(
        needs_layout_passes=False, use_tc_tiling_on_sc=True),
    out_type=(jax.ShapeDtypeStruct((ROWS, 128), jnp.float32),
              jax.ShapeDtypeStruct((ROWS, 128), jnp.float32)),
    scratch_types=[
        pltpu.VMEM((B,), jnp.int32),
        pltpu.VMEM((MCAP,), jnp.int32),
        pltpu.VMEM((MCAP,), jnp.int32),
        pltpu.VMEM((HCAP,), jnp.int32),
        pltpu.VMEM((HCAP,), jnp.int32),
        pltpu.VMEM((256,), jnp.int32),
        pltpu.VMEM((256,), jnp.int32),
        pltpu.VMEM((256,), jnp.int32),
        pltpu.VMEM((272,), jnp.int32),
        pltpu.VMEM((D, 128), jnp.float32),
        pltpu.VMEM((D, 128), jnp.float32),
        pltpu.VMEM((D, 128), jnp.float32),
        pltpu.VMEM((D, 128), jnp.float32),
        pltpu.VMEM((D, 128), jnp.float32),
        pltpu.VMEM((D, 128), jnp.float32),
        pltpu.VMEM((16, 128), jnp.float32),
        pltpu.VMEM((16,), jnp.int32),
        pltpu.SemaphoreType.DMA,
        pltpu.SemaphoreType.DMA,
        pltpu.SemaphoreType.DMA,
        pltpu.SemaphoreType.DMA,
        pltpu.SemaphoreType.DMA,
        pltpu.SemaphoreType.DMA,
        pltpu.SemaphoreType.DMA,
    ],
)
def _sweep_sc(uids_hbm, iids_hbm, pu_hbm, pv_hbm, ug_hbm, vg_hbm,
              idb, moff, mpos, hid, hpo, soff, spos, wcnts, wlist, w0b, w1b, w2b, w3b, w4b, w5b,
              stage, posb, sem0, sem1, sem2, sem3, sem4, sem5, ssem):
    wid = lax.axis_index("s") * NC + lax.axis_index("c")
    W0 = jnp.where(wid < 5, 245 * wid, 244 * wid + 5)
    NWIN = jnp.where(wid < 5, 245, 244)
    lanes = lax.iota(jnp.int32, L)
    lane0 = lanes == 0
    wbufs = (w0b, w1b, w2b, w3b, w4b, w5b)
    wsems = (sem0, sem1, sem2, sem3, sem4, sem5)

    zero16 = jnp.zeros((L,), jnp.float32)
    for r in range(16):
        for c in range(4, 8):
            plsc.store_scatter(
                stage, [jnp.full((L,), r, jnp.int32),
                        c * L + lanes], zero16)
    posb[...] = jnp.full((L,), DUMP, jnp.int32)

    def do_table(tab_hbm, ids_hbm, out_hbm):
        pltpu.sync_copy(ids_hbm, idb)
        lo128 = W0 * 128
        hi128 = (W0 + NWIN) * 128

        def a_body(g, cnt):
            ids16 = idb[pl.ds(g * L, L)]
            m = (ids16 >= lo128) & (ids16 < hi128)
            plsc.store_compressed(moff.at[pl.ds(cnt, L)], ids16, mask=m)
            plsc.store_compressed(
                mpos.at[pl.ds(cnt, L)], g * L + lanes, mask=m)
            c = plsc.all_reduce_population_count(m)[0]
            return jnp.minimum(cnt + c, MCAP - L)

        cnt = lax.fori_loop(0, B // L, a_body, jnp.int32(0))
        ngroups = (cnt + L - 1) // L

        zeros16i = jnp.zeros((L,), jnp.int32)
        for i in range(16):
            wcnts[pl.ds(i * L, L)] = zeros16i
        ones16 = jnp.full((L,), 1, jnp.int32)

        def h_body(gi, _):
            i16 = moff[pl.ds(gi * L, L)]
            valid = (gi * L + lanes) < cnt
            plsc.addupdate_scatter(
                wcnts, [(i16 >> 7) - W0], ones16, mask=valid)
            return 0

        lax.fori_loop(0, ngroups, h_body, 0)

        def l_body(gi, wc):
            widx16 = gi * L + lanes
            c16 = wcnts[pl.ds(gi * L, L)]
            m = (c16 > 0) & (widx16 < NWIN)
            plsc.store_compressed(wlist.at[pl.ds(wc, L)], widx16, mask=m)
            return wc + plsc.all_reduce_population_count(m)[0]

        wcount = lax.fori_loop(0, 16, l_body, jnp.int32(0))

        def entry(j):
            jj = jnp.maximum(jnp.minimum(j, wcount - 1), 0)
            return wlist[pl.ds(jj, L)][0]

        def fire_entry(j, slot):
            woff = jnp.minimum(W0 + entry(j), LASTWIN) * 128
            woff = pl.multiple_of(woff, 128)
            return pltpu.async_copy(
                tab_hbm.at[:, pl.ds(woff, 128)], wbufs[slot], wsems[slot])

        for s in range(6):
            fire_entry(jnp.int32(s), s)

        def flush_stage(ss):
            pv = posb[...]
            pltpu.async_copy(stage, out_hbm.at[pv], ssem).wait()
            posb[...] = jnp.full((L,), DUMP, jnp.int32)
            return ss

        def super_filter(w4):
            e0 = w4 * 6
            sw_lo = W0 + entry(e0)
            sw_hi = W0 + entry(e0 + 17)

            def sscan(gi, sc):
                i16 = moff[pl.ds(gi * L, L)]
                p16 = mpos[pl.ds(gi * L, L)]
                valid = (gi * L + lanes) < cnt
                wg = i16 >> 7
                m = (wg >= sw_lo) & (wg <= sw_hi) & valid
                plsc.store_compressed(soff.at[pl.ds(sc, L)], i16, mask=m)
                plsc.store_compressed(spos.at[pl.ds(sc, L)], p16, mask=m)
                c = plsc.all_reduce_population_count(m)[0]
                return jnp.minimum(sc + c, 256 - L)

            return lax.fori_loop(0, ngroups, sscan, jnp.int32(0))

        def process_window(j, k, ss, scnt):
            g = W0 + entry(j)

            def scan(gi, hcnt):
                i16 = soff[pl.ds(gi * L, L)]
                p16 = spos[pl.ds(gi * L, L)]
                valid = (gi * L + lanes) < scnt
                m = ((i16 >> 7) == g) & valid
                plsc.store_compressed(hid.at[pl.ds(hcnt, L)], i16, mask=m)
                plsc.store_compressed(hpo.at[pl.ds(hcnt, L)], p16, mask=m)
                c = plsc.all_reduce_population_count(m)[0]
                return jnp.minimum(hcnt + c, HCAP - L)

            sgroups = (scnt + L - 1) // L
            hcnt = lax.fori_loop(0, sgroups, scan, jnp.int32(0))

            def per_id(i, ss):
                idv = hid[pl.ds(i, L)][0]
                pos = hpo[pl.ds(i, L)][0]
                xs = jnp.full((L,), idv & 127, jnp.int32)
                slot = ss % 16
                rows16 = jnp.full((L,), slot, jnp.int32)
                for c in range(4):
                    vals = plsc.load_gather(wbufs[k], [c * L + lanes, xs])
                    plsc.store_scatter(stage, [rows16, c * L + lanes], vals)
                plsc.store_scatter(
                    posb, [rows16], jnp.full((L,), pos, jnp.int32),
                    mask=lane0)
                return lax.cond(slot == 15, flush_stage,
                                lambda s: s, ss) + 1

            return lax.fori_loop(0, hcnt, per_id, ss)

        def w_body(w4, carry):
            ss, scnt = carry
            scnt = lax.cond(w4 % 3 == 0,
                            lambda: super_filter(w4),
                            lambda: scnt)
            for k in range(6):
                j = w4 * 6 + k
                pltpu.make_async_copy(
                    tab_hbm.at[:, pl.ds(0, 128)], wbufs[k],
                    wsems[k]).wait()
                ss = lax.cond(
                    j < wcount,
                    lambda s, j=j, k=k: process_window(j, k, s, scnt),
                    lambda s: s, ss)
                fire_entry(j + 6, k)
            return ss, scnt

        nouter = (wcount + 5) // 6
        ss, _ = lax.fori_loop(
            0, nouter, w_body, (jnp.int32(0), jnp.int32(0)))
        for k in range(6):
            pltpu.make_async_copy(
                tab_hbm.at[:, pl.ds(0, 128)], wbufs[k], wsems[k]).wait()
        flush_stage(ss)

    do_table(pu_hbm, uids_hbm, ug_hbm)
    do_table(pv_hbm, iids_hbm, vg_hbm)


def _dot_body(u_ref, v_ref, w_ref, b_ref, o_ref):
    h = u_ref[...] * v_ref[...]
    o_ref[...] = lax.dot_general(
        h, w_ref[...], (((1,), (0,)), ((), ())),
        preferred_element_type=jnp.float32) + b_ref[...]


_dot_tc = pl.pallas_call(
    _dot_body,
    grid=(4,),
    in_specs=[
        pl.BlockSpec((4096, 128), lambda i: (i, 0)),
        pl.BlockSpec((4096, 128), lambda i: (i, 0)),
        pl.BlockSpec((128, 1), lambda i: (0, 0)),
        pl.BlockSpec((1, 1), lambda i: (0, 0)),
    ],
    out_specs=pl.BlockSpec((4096, 1), lambda i: (i, 0)),
    out_shape=jax.ShapeDtypeStruct((B, 1), jnp.float32),
)


def kernel(user_ids, item_ids, user_table, item_table, W, b):
    uids = user_ids.astype(jnp.int32)
    iids = item_ids.astype(jnp.int32)
    ug, vg = _sweep_sc(uids, iids, user_table.T, item_table.T)
    wpad = jnp.zeros((128, 1), jnp.float32).at[:D, 0].set(W[:, 0])
    out2 = _dot_tc(ug, vg, wpad, b.reshape(1, 1))
    return out2[:, 0]
